# baseline (speedup 1.0000x reference)
import functools

import jax
import jax.numpy as jnp
from jax import lax
from jax.experimental import pallas as pl
from jax.experimental.pallas import tpu as pltpu
from jax.experimental.pallas import tpu_sc as plsc

N = 10000
D = 128
E = 320000
G = 64
NCLS = 1000

NC, NS = 2, 16
NW = NC * NS
CH = 128
NCHUNK = 80
E_PAD = NW * NCHUNK * CH
N_PAD = 10112
ROWS_PT = N_PAD // NS

_MESH = plsc.VectorSubcoreMesh(
    core_axis_name="c", subcore_axis_name="s", num_cores=NC, num_subcores=NS
)


def _spmm_body(hs_hbm, src_hbm, dst_hbm, zeros_hbm, out_hbm,
               src_c, dst_c, buf, acc, sg):
    c = lax.axis_index("c")
    s = lax.axis_index("s")
    wid = s * NC + c
    r0 = s * ROWS_PT
    pltpu.sync_copy(zeros_hbm.at[pl.ds(r0, ROWS_PT)], acc.at[pl.ds(r0, ROWS_PT)])
    plsc.subcore_barrier()

    @pl.loop(0, NCHUNK)
    def _(j):
        base = (wid * NCHUNK + j) * CH
        pltpu.sync_copy(src_hbm.at[pl.ds(base, CH)], src_c)
        pltpu.sync_copy(dst_hbm.at[pl.ds(base, CH)], dst_c)
        pltpu.async_copy(hs_hbm.at[src_c], buf, sg).wait()
        pltpu.sync_copy(buf, acc.at[dst_c], add=True)

    plsc.subcore_barrier()
    pltpu.sync_copy(acc.at[pl.ds(r0, ROWS_PT)], out_hbm.at[c, pl.ds(r0, ROWS_PT)])


_spmm_call = pl.kernel(
    _spmm_body,
    out_type=jax.ShapeDtypeStruct((NC, N_PAD, D), jnp.float32),
    mesh=_MESH,
    scratch_types=[
        pltpu.VMEM((CH,), jnp.int32),
        pltpu.VMEM((CH,), jnp.int32),
        pltpu.VMEM((CH, D), jnp.float32),
        pltpu.VMEM_SHARED((N_PAD, D), jnp.float32),
        pltpu.SemaphoreType.DMA,
    ],
)


_RB = 1000


def _dinv(dega_ref, degb_ref):
    deg = dega_ref[:, :1] + degb_ref[:, :1] + 1.0
    return lax.rsqrt(deg)


def _prep_body(x_ref, w1_ref, dega_ref, degb_ref, h1s_ref):
    h1 = jnp.dot(x_ref[...], w1_ref[...], preferred_element_type=jnp.float32)
    h1s_ref[...] = h1 * _dinv(dega_ref, degb_ref)


def _mid_body(sa_ref, sb_ref, h1s_ref, dega_ref, degb_ref, b1_ref, w2_ref,
              h2s_ref):
    dinv = _dinv(dega_ref, degb_ref)
    z1 = jnp.maximum(
        dinv * (sa_ref[...] + sb_ref[...] + h1s_ref[...]) + b1_ref[...], 0.0)
    h2 = jnp.dot(z1, w2_ref[...], preferred_element_type=jnp.float32)
    h2s_ref[...] = h2 * dinv


def _final_body(sa_ref, sb_ref, h2s_ref, dega_ref, degb_ref, b2_ref,
                bidx_ref, wout_ref, bout_ref, out_ref, pool_acc, cnt_acc):
    i = pl.program_id(0)
    dinv = _dinv(dega_ref, degb_ref)
    z2 = jnp.maximum(
        dinv * (sa_ref[...] + sb_ref[...] + h2s_ref[...]) + b2_ref[...], 0.0)
    gids = lax.broadcasted_iota(jnp.int32, (_RB, G), 1).astype(jnp.float32)
    oh = (bidx_ref[...] == gids).astype(jnp.float32)

    @pl.when(i == 0)
    def _():
        pool_acc[...] = jnp.zeros((G, D), jnp.float32)
        cnt_acc[...] = jnp.zeros((G, D), jnp.float32)

    dn = (((0,), (0,)), ((), ()))
    pool_acc[...] += lax.dot_general(oh, z2, dn,
                                     preferred_element_type=jnp.float32)
    cnt_acc[...] += lax.dot_general(oh, jnp.ones((_RB, D), jnp.float32), dn,
                                    preferred_element_type=jnp.float32)

    @pl.when(i == pl.num_programs(0) - 1)
    def _():
        pooled = pool_acc[...] / jnp.maximum(cnt_acc[...], 1.0)
        out_ref[...] = jnp.dot(pooled, wout_ref[...],
                               preferred_element_type=jnp.float32) + bout_ref[...]


_prep_call = pl.pallas_call(
    _prep_body,
    grid=(N // _RB,),
    in_specs=[
        pl.BlockSpec((_RB, D), lambda i: (i, 0)),
        pl.BlockSpec((D, D), lambda i: (0, 0)),
        pl.BlockSpec((_RB, D), lambda i: (i, 0)),
        pl.BlockSpec((_RB, D), lambda i: (i, 0)),
    ],
    out_specs=pl.BlockSpec((_RB, D), lambda i: (i, 0)),
    out_shape=jax.ShapeDtypeStruct((N, D), jnp.float32),
)

_mid_call = pl.pallas_call(
    _mid_body,
    grid=(N // _RB,),
    in_specs=[
        pl.BlockSpec((_RB, D), lambda i: (i, 0)),
        pl.BlockSpec((_RB, D), lambda i: (i, 0)),
        pl.BlockSpec((_RB, D), lambda i: (i, 0)),
        pl.BlockSpec((_RB, D), lambda i: (i, 0)),
        pl.BlockSpec((_RB, D), lambda i: (i, 0)),
        pl.BlockSpec((1, D), lambda i: (0, 0)),
        pl.BlockSpec((D, D), lambda i: (0, 0)),
    ],
    out_specs=pl.BlockSpec((_RB, D), lambda i: (i, 0)),
    out_shape=jax.ShapeDtypeStruct((N, D), jnp.float32),
)

_final_call = pl.pallas_call(
    _final_body,
    grid=(N // _RB,),
    in_specs=[
        pl.BlockSpec((_RB, D), lambda i: (i, 0)),
        pl.BlockSpec((_RB, D), lambda i: (i, 0)),
        pl.BlockSpec((_RB, D), lambda i: (i, 0)),
        pl.BlockSpec((_RB, D), lambda i: (i, 0)),
        pl.BlockSpec((_RB, D), lambda i: (i, 0)),
        pl.BlockSpec((1, D), lambda i: (0, 0)),
        pl.BlockSpec((_RB, 1), lambda i: (i, 0)),
        pl.BlockSpec((D, NCLS), lambda i: (0, 0)),
        pl.BlockSpec((1, NCLS), lambda i: (0, 0)),
    ],
    out_specs=pl.BlockSpec((G, NCLS), lambda i: (0, 0)),
    out_shape=jax.ShapeDtypeStruct((G, NCLS), jnp.float32),
    scratch_shapes=[
        pltpu.VMEM((G, D), jnp.float32),
        pltpu.VMEM((G, D), jnp.float32),
    ],
)


@jax.jit
def kernel(x, edge_index, batch_index, W1, b1, W2, b2, W_out, b_out):
    src = edge_index[0].astype(jnp.int32)
    dst = edge_index[1].astype(jnp.int32)
    pad_n = E_PAD - E
    src1d = jnp.concatenate([src, jnp.zeros((pad_n,), jnp.int32)])
    pad_dst = N + jnp.arange(pad_n, dtype=jnp.int32) % (N_PAD - N)
    dst1d = jnp.concatenate([dst, pad_dst])

    zeros_acc = jnp.zeros((N_PAD, D), jnp.float32)
    ones_nd = jnp.ones((N, D), jnp.float32)
    bidx = batch_index.astype(jnp.float32).reshape(N, 1)

    deg = _spmm_call(ones_nd, src1d, dst1d, zeros_acc)
    dega, degb = deg[0], deg[1]

    h1s = _prep_call(x, W1, dega, degb)
    s1 = _spmm_call(h1s, src1d, dst1d, zeros_acc)
    h2s = _mid_call(s1[0], s1[1], h1s, dega, degb, b1.reshape(1, D), W2)
    s2 = _spmm_call(h2s, src1d, dst1d, zeros_acc)
    out = _final_call(s2[0], s2[1], h2s, dega, degb, b2.reshape(1, D),
                      bidx, W_out, b_out.reshape(1, NCLS))
    return out
